# initial kernel scaffold (unmeasured)
import jax
import jax.numpy as jnp
from jax import lax
from jax.experimental import pallas as pl
from jax.experimental.pallas import tpu as pltpu


def kernel(
    x,
):
    def body(*refs):
        pass

    out_shape = jax.ShapeDtypeStruct(..., jnp.float32)
    return pl.pallas_call(body, out_shape=out_shape)(...)


# baseline (device time: 54095 ns/iter reference)
import jax
import jax.numpy as jnp
from jax import lax
from jax.experimental import pallas as pl
from jax.experimental.pallas import tpu as pltpu


def kernel(x):
    m_per, n = x.shape

    def body(x_ref, out_ref, send_sem, recv_sem):
        my_x = lax.axis_index("x")
        my_y = lax.axis_index("y")
        my_z = lax.axis_index("z")
        partner = (1 - my_x, my_y, my_z)

        barrier_sem = pltpu.get_barrier_semaphore()
        pl.semaphore_signal(
            barrier_sem, inc=1,
            device_id=partner, device_id_type=pl.DeviceIdType.MESH,
        )
        pl.semaphore_wait(barrier_sem, 1)

        def branch(xi):
            def _():
                lo = xi * m_per
                out_ref[pl.ds(lo, m_per), :] = x_ref[...]
                rdma = pltpu.make_async_remote_copy(
                    src_ref=x_ref,
                    dst_ref=out_ref.at[pl.ds(lo, m_per), :],
                    send_sem=send_sem,
                    recv_sem=recv_sem,
                    device_id=partner,
                    device_id_type=pl.DeviceIdType.MESH,
                )
                rdma.start()
                rdma.wait()
            return _

        pl.when(my_x == 0)(branch(0))
        pl.when(my_x == 1)(branch(1))

    return pl.pallas_call(
        body,
        out_shape=jax.ShapeDtypeStruct((2 * m_per, n), x.dtype),
        in_specs=[pl.BlockSpec(memory_space=pltpu.VMEM)],
        out_specs=pl.BlockSpec(memory_space=pltpu.VMEM),
        scratch_shapes=[
            pltpu.SemaphoreType.DMA,
            pltpu.SemaphoreType.DMA,
        ],
        compiler_params=pltpu.CompilerParams(collective_id=0),
    )(x)


# device time: 36837 ns/iter; 1.4685x vs baseline; 1.4685x over previous
import jax
import jax.numpy as jnp
from jax import lax
from jax.experimental import pallas as pl
from jax.experimental.pallas import tpu as pltpu

C = 8


def kernel(x):
    m_per, n = x.shape
    half = m_per // 2
    rc = half // C

    def body(x_ref, out_ref, x_send, x_recv, y_send, y_recv):
        my_x = lax.axis_index("x")
        my_y = lax.axis_index("y")
        my_z = lax.axis_index("z")
        xp = (1 - my_x, my_y, my_z)
        yp = (my_x, 1 - my_y, my_z)

        barrier_sem = pltpu.get_barrier_semaphore()
        pl.semaphore_signal(
            barrier_sem, inc=1, device_id=xp,
            device_id_type=pl.DeviceIdType.MESH,
        )
        pl.semaphore_signal(
            barrier_sem, inc=1, device_id=yp,
            device_id_type=pl.DeviceIdType.MESH,
        )
        pl.semaphore_wait(barrier_sem, 2)

        my_half = my_y * half
        dst_base = my_x * m_per + my_half
        foreign = (1 - my_x) * m_per

        x_rdmas = []
        for c in range(C):
            r = pltpu.make_async_remote_copy(
                src_ref=x_ref.at[pl.ds(my_half + c * rc, rc), :],
                dst_ref=out_ref.at[pl.ds(dst_base + c * rc, rc), :],
                send_sem=x_send.at[c],
                recv_sem=x_recv.at[c],
                device_id=xp,
                device_id_type=pl.DeviceIdType.MESH,
            )
            r.start()
            x_rdmas.append(r)

        out_ref[pl.ds(my_x * m_per, m_per), :] = x_ref[...]

        y_rdmas = []
        for c in range(C):
            x_rdmas[c].wait_recv()
            fwd = foreign + my_half + c * rc
            r = pltpu.make_async_remote_copy(
                src_ref=out_ref.at[pl.ds(fwd, rc), :],
                dst_ref=out_ref.at[pl.ds(fwd, rc), :],
                send_sem=y_send.at[c],
                recv_sem=y_recv.at[c],
                device_id=yp,
                device_id_type=pl.DeviceIdType.MESH,
            )
            r.start()
            y_rdmas.append(r)

        for c in range(C):
            y_rdmas[c].wait_recv()
        for c in range(C):
            x_rdmas[c].wait_send()
            y_rdmas[c].wait_send()

    return pl.pallas_call(
        body,
        out_shape=jax.ShapeDtypeStruct((2 * m_per, n), x.dtype),
        in_specs=[pl.BlockSpec(memory_space=pltpu.VMEM)],
        out_specs=pl.BlockSpec(memory_space=pltpu.VMEM),
        scratch_shapes=[
            pltpu.SemaphoreType.DMA((C,)),
            pltpu.SemaphoreType.DMA((C,)),
            pltpu.SemaphoreType.DMA((C,)),
            pltpu.SemaphoreType.DMA((C,)),
        ],
        compiler_params=pltpu.CompilerParams(collective_id=0),
    )(x)
